# baseline (speedup 1.0000x reference)
import functools

import jax
import jax.numpy as jnp
from jax import lax
from jax.experimental import pallas as pl
from jax.experimental.pallas import tpu as pltpu
from jax.experimental.pallas import tpu_sc as plsc

N = 10000
D_IN = 128
H1 = 8
C1 = 8
D1 = H1 * C1
D2 = 128

NT = 10240
NW = 32
NJ = 2
B_C = NJ * 128
E_TOT = 320000 + N
C_W = 42
EPAD = NW * B_C * C_W
RPT = NT // 16

_mesh = plsc.VectorSubcoreMesh(core_axis_name="c", subcore_axis_name="s",
                               num_cores=2, num_subcores=16)
_sc_params = pltpu.CompilerParams(use_tc_tiling_on_sc=False)


def _f32(shape):
    return jax.ShapeDtypeStruct(shape, jnp.float32)


def _wait(src, dst, sem):
    pltpu.make_async_copy(src, dst, sem).wait()


@functools.partial(
    pl.kernel,
    out_type=(_f32((EPAD, 16)), _f32((2, NT, 16))),
    mesh=_mesh,
    compiler_params=_sc_params,
    scratch_types=[
        pltpu.VMEM((2 * NJ, 128), jnp.int32),
        pltpu.VMEM((2 * NJ, 128), jnp.int32),
        pltpu.VMEM((2, B_C, 16), jnp.float32),
        pltpu.VMEM((2, B_C, 16), jnp.float32),
        pltpu.VMEM_SHARED((NT, 16), jnp.float32),
        pltpu.SemaphoreType.DMA,
        pltpu.SemaphoreType.DMA,
    ],
)
def _sc_pass_a(src_hbm, dst_hbm, as_hbm, ad_hbm, ex_hbm, den_hbm,
               src_v, dst_v, as_v, ad_v, den_sh, sem0, sem1):
    cid = lax.axis_index("c")
    sid = lax.axis_index("s")
    wid = cid * 16 + sid
    sems = (sem0, sem1)

    z16 = jnp.zeros((16,), jnp.float32)

    def _zb(i, _):
        as_v[0, i, :] = z16
        return 0

    lax.fori_loop(0, B_C, _zb, 0, unroll=8)
    for r in range(RPT // B_C):
        pltpu.sync_copy(as_v.at[0],
                        den_sh.at[pl.ds(sid * RPT + r * B_C, B_C)])
    plsc.subcore_barrier()

    base_w = wid * (C_W * B_C)

    def _load_and_gather(t, b):
        base = base_w + t * B_C
        for j in range(NJ):
            pltpu.sync_copy(src_hbm.at[pl.ds(base + j * 128, 128)],
                            src_v.at[b * NJ + j])
            pltpu.sync_copy(dst_hbm.at[pl.ds(base + j * 128, 128)],
                            dst_v.at[b * NJ + j])
        for j in range(NJ):
            pltpu.async_copy(as_hbm.at[src_v.at[b * NJ + j]],
                             as_v.at[b, pl.ds(j * 128, 128)], sems[b])
            pltpu.async_copy(ad_hbm.at[dst_v.at[b * NJ + j]],
                             ad_v.at[b, pl.ds(j * 128, 128)], sems[b])

    def _wait_gathers(b):
        for j in range(NJ):
            _wait(as_hbm.at[src_v.at[b * NJ + j]],
                  as_v.at[b, pl.ds(j * 128, 128)], sems[b])
            _wait(ad_hbm.at[dst_v.at[b * NJ + j]],
                  ad_v.at[b, pl.ds(j * 128, 128)], sems[b])

    _load_and_gather(0, 0)

    def _outer(tt, _):
        for b in range(2):
            t = 2 * tt + b

            @pl.when(t + 1 < C_W)
            def _():
                _load_and_gather(t + 1, b ^ 1)

            _wait_gathers(b)

            def _edge(i, _):
                e = as_v[b, i, :] + ad_v[b, i, :]
                e = jnp.where(e > 0, e, 0.2 * e)
                as_v[b, i, :] = jnp.exp(e)
                return 0

            lax.fori_loop(0, B_C, _edge, 0, unroll=16)
            base = base_w + t * B_C
            pltpu.sync_copy(as_v.at[b], ex_hbm.at[pl.ds(base, B_C)])
            for j in range(NJ):
                pltpu.sync_copy(as_v.at[b, pl.ds(j * 128, 128)],
                                den_sh.at[dst_v.at[b * NJ + j]], add=True)
        return 0

    lax.fori_loop(0, C_W // 2, _outer, 0)

    plsc.subcore_barrier()
    pltpu.sync_copy(den_sh.at[pl.ds(sid * RPT, RPT)],
                    den_hbm.at[cid, pl.ds(sid * RPT, RPT)])


def _make_sc_pass_b(D, expand_pairs, nj):
    b_c = nj * 128
    c_w = (C_W * B_C) // b_c

    @functools.partial(
        pl.kernel,
        out_type=_f32((2, NT, D)),
        mesh=_mesh,
        compiler_params=_sc_params,
        scratch_types=[
            pltpu.VMEM((2 * nj, 128), jnp.int32),
            pltpu.VMEM((2 * nj, 128), jnp.int32),
            pltpu.VMEM((2, b_c, 16), jnp.float32),
            pltpu.VMEM((2, b_c, 16), jnp.float32),
            pltpu.VMEM((2, b_c, D), jnp.float32),
            pltpu.VMEM_SHARED((NT, D), jnp.float32),
            pltpu.SemaphoreType.DMA,
            pltpu.SemaphoreType.DMA,
        ],
    )
    def _sc_pass_b(src_hbm, dst_hbm, ex_hbm, h_hbm, den_hbm, out_hbm,
                   src_v, dst_v, ex_v, den_v, h_v, out_sh,
                   sem0, sem1):
        cid = lax.axis_index("c")
        sid = lax.axis_index("s")
        wid = cid * 16 + sid
        sems = (sem0, sem1)

        z16 = jnp.zeros((16,), jnp.float32)

        def _zb(i, _):
            for k in range(D // 16):
                h_v[0, i, pl.ds(k * 16, 16)] = z16
            return 0

        lax.fori_loop(0, 128, _zb, 0, unroll=8)
        for r in range(RPT // 128):
            pltpu.sync_copy(h_v.at[0, pl.ds(0, 128)],
                            out_sh.at[pl.ds(sid * RPT + r * 128, 128)])
        plsc.subcore_barrier()

        if expand_pairs:
            lane_hi = lax.iota(jnp.int32, 16) >= 8

        base_w = wid * (c_w * b_c)

        def _load_and_gather(t, b):
            base = base_w + t * b_c
            for j in range(nj):
                pltpu.sync_copy(src_hbm.at[pl.ds(base + j * 128, 128)],
                                src_v.at[b * nj + j])
                pltpu.sync_copy(dst_hbm.at[pl.ds(base + j * 128, 128)],
                                dst_v.at[b * nj + j])
            pltpu.async_copy(ex_hbm.at[pl.ds(base, b_c)], ex_v.at[b],
                             sems[b])
            for j in range(nj):
                pltpu.async_copy(h_hbm.at[src_v.at[b * nj + j]],
                                 h_v.at[b, pl.ds(j * 128, 128)], sems[b])
                pltpu.async_copy(den_hbm.at[dst_v.at[b * nj + j]],
                                 den_v.at[b, pl.ds(j * 128, 128)], sems[b])

        def _wait_gathers(t, b):
            base = base_w + t * b_c
            _wait(ex_hbm.at[pl.ds(base, b_c)], ex_v.at[b], sems[b])
            for j in range(nj):
                _wait(h_hbm.at[src_v.at[b * nj + j]],
                      h_v.at[b, pl.ds(j * 128, 128)], sems[b])
                _wait(den_hbm.at[dst_v.at[b * nj + j]],
                      den_v.at[b, pl.ds(j * 128, 128)], sems[b])

        _load_and_gather(0, 0)

        def _outer(tt, _):
            for b in range(2):
                t = 2 * tt + b

                @pl.when(t + 1 < c_w)
                def _():
                    _load_and_gather(t + 1, b ^ 1)

                _wait_gathers(t, b)

                def _edge(i, _):
                    alpha = ex_v[b, i, :] / (den_v[b, i, :] + 1e-16)
                    for k in range(D // 16):
                        if expand_pairs:
                            a = jnp.where(lane_hi, alpha[2 * k + 1],
                                          alpha[2 * k])
                        else:
                            a = alpha
                        h_v[b, i, pl.ds(k * 16, 16)] = (
                            h_v[b, i, pl.ds(k * 16, 16)] * a)
                    return 0

                lax.fori_loop(0, b_c, _edge, 0, unroll=8)
                for j in range(nj):
                    pltpu.sync_copy(h_v.at[b, pl.ds(j * 128, 128)],
                                    out_sh.at[dst_v.at[b * nj + j]],
                                    add=True)
            return 0

        lax.fori_loop(0, c_w // 2, _outer, 0)

        plsc.subcore_barrier()
        pltpu.sync_copy(out_sh.at[pl.ds(sid * RPT, RPT)],
                        out_hbm.at[cid, pl.ds(sid * RPT, RPT)])

    return _sc_pass_b


_sc_pass_b1 = _make_sc_pass_b(D1, True, 2)
_sc_pass_b2 = _make_sc_pass_b(D2, False, 1)


_BR = 256


def _proj1_body(x_ref, w_ref, was_ref, wad_ref, h_ref, as_ref, ad_ref):
    x = x_ref[...]
    h_ref[...] = jnp.dot(x, w_ref[...], preferred_element_type=jnp.float32)
    as_ref[...] = jnp.dot(x, was_ref[...], preferred_element_type=jnp.float32)
    ad_ref[...] = jnp.dot(x, wad_ref[...], preferred_element_type=jnp.float32)


def _proj1(xp, W1, Was16, Wad16):
    return pl.pallas_call(
        _proj1_body,
        grid=(NT // _BR,),
        in_specs=[
            pl.BlockSpec((_BR, D_IN), lambda i: (i, 0)),
            pl.BlockSpec((D_IN, D1), lambda i: (0, 0)),
            pl.BlockSpec((D_IN, 16), lambda i: (0, 0)),
            pl.BlockSpec((D_IN, 16), lambda i: (0, 0)),
        ],
        out_specs=[
            pl.BlockSpec((_BR, D1), lambda i: (i, 0)),
            pl.BlockSpec((_BR, 16), lambda i: (i, 0)),
            pl.BlockSpec((_BR, 16), lambda i: (i, 0)),
        ],
        out_shape=[_f32((NT, D1)), _f32((NT, 16)), _f32((NT, 16))],
    )(xp, W1, Was16, Wad16)


def _proj2_body(p0_ref, p1_ref, b_ref, w_ref, was_ref, wad_ref,
                h_ref, as_ref, ad_ref):
    h1e = p0_ref[0] + p1_ref[0] + b_ref[...]
    h1e = jnp.where(h1e > 0, h1e, jnp.exp(h1e) - 1.0)
    h_ref[...] = jnp.dot(h1e, w_ref[...], preferred_element_type=jnp.float32)
    as_ref[...] = jnp.dot(h1e, was_ref[...],
                          preferred_element_type=jnp.float32)
    ad_ref[...] = jnp.dot(h1e, wad_ref[...],
                          preferred_element_type=jnp.float32)


def _proj2(out1p, b1, W2, Was16, Wad16):
    return pl.pallas_call(
        _proj2_body,
        grid=(NT // _BR,),
        in_specs=[
            pl.BlockSpec((1, _BR, D1), lambda i: (0, i, 0)),
            pl.BlockSpec((1, _BR, D1), lambda i: (1, i, 0)),
            pl.BlockSpec((1, D1), lambda i: (0, 0)),
            pl.BlockSpec((D1, D2), lambda i: (0, 0)),
            pl.BlockSpec((D1, 16), lambda i: (0, 0)),
            pl.BlockSpec((D1, 16), lambda i: (0, 0)),
        ],
        out_specs=[
            pl.BlockSpec((_BR, D2), lambda i: (i, 0)),
            pl.BlockSpec((_BR, 16), lambda i: (i, 0)),
            pl.BlockSpec((_BR, 16), lambda i: (i, 0)),
        ],
        out_shape=[_f32((NT, D2)), _f32((NT, 16)), _f32((NT, 16))],
    )(out1p, out1p, b1, W2, Was16, Wad16)


def _comb_body(p0_ref, p1_ref, o_ref):
    o_ref[...] = p0_ref[0] + p1_ref[0]


def _combine(parts):
    D = parts.shape[-1]
    return pl.pallas_call(
        _comb_body,
        grid=(NT // _BR,),
        in_specs=[
            pl.BlockSpec((1, _BR, D), lambda i: (0, i, 0)),
            pl.BlockSpec((1, _BR, D), lambda i: (1, i, 0)),
        ],
        out_specs=pl.BlockSpec((_BR, D), lambda i: (i, 0)),
        out_shape=_f32((NT, D)),
    )(parts, parts)


def _final_body(p0_ref, p1_ref, b_ref, o_ref):
    o_ref[...] = p0_ref[0] + p1_ref[0] + b_ref[...]


def _final(parts, b2):
    return pl.pallas_call(
        _final_body,
        grid=(NT // _BR,),
        in_specs=[
            pl.BlockSpec((1, _BR, D2), lambda i: (0, i, 0)),
            pl.BlockSpec((1, _BR, D2), lambda i: (1, i, 0)),
            pl.BlockSpec((1, D2), lambda i: (0, 0)),
        ],
        out_specs=pl.BlockSpec((_BR, D2), lambda i: (i, 0)),
        out_shape=_f32((NT, D2)),
    )(parts, parts, b2)


def kernel(x, edge_index, W1, a_src1, a_dst1, b1, W2, a_src2, a_dst2, b2):
    loops = jnp.arange(N, dtype=edge_index.dtype)
    src = jnp.concatenate([edge_index[0], loops])
    dst = jnp.concatenate([edge_index[1], loops])
    pad = EPAD - E_TOT
    sent = (N + jnp.arange(pad, dtype=jnp.int32) % (NT - N)).astype(
        edge_index.dtype)
    src = jnp.concatenate([src, sent])
    dst = jnp.concatenate([dst, sent])

    xp = jnp.pad(x, ((0, NT - N), (0, 0)))

    Was1 = (W1.reshape(D_IN, H1, C1) * a_src1[None]).sum(-1)
    Wad1 = (W1.reshape(D_IN, H1, C1) * a_dst1[None]).sum(-1)
    Was1_16 = jnp.concatenate([Was1, Was1], axis=1)
    Wad1_16 = jnp.concatenate([Wad1, Wad1], axis=1)
    Was2_16 = jnp.tile((W2 @ a_src2[0])[:, None], (1, 16))
    Wad2_16 = jnp.tile((W2 @ a_dst2[0])[:, None], (1, 16))

    h1, as1, ad1 = _proj1(xp, W1, Was1_16, Wad1_16)
    ex1, den1p = _sc_pass_a(src, dst, as1, ad1)
    den1 = _combine(den1p)
    out1p = _sc_pass_b1(src, dst, ex1, h1, den1)

    h2, as2, ad2 = _proj2(out1p, b1[None, :], W2, Was2_16, Wad2_16)
    ex2, den2p = _sc_pass_a(src, dst, as2, ad2)
    den2 = _combine(den2p)
    out2p = _sc_pass_b2(src, dst, ex2, h2, den2)

    out = _final(out2p, b2[None, :])
    return out[:N]


# speedup vs baseline: 1.1757x; 1.1757x over previous
import functools

import jax
import jax.numpy as jnp
from jax import lax
from jax.experimental import pallas as pl
from jax.experimental.pallas import tpu as pltpu
from jax.experimental.pallas import tpu_sc as plsc

N = 10000
D_IN = 128
H1 = 8
C1 = 8
D1 = H1 * C1
D2 = 128

NT = 10240
NW = 32
NJ = 2
B_C = NJ * 128
E_TOT = 320000 + N
C_W = 42
EPAD = NW * B_C * C_W
RPT = NT // 16

_mesh = plsc.VectorSubcoreMesh(core_axis_name="c", subcore_axis_name="s",
                               num_cores=2, num_subcores=16)
_sc_params = pltpu.CompilerParams(use_tc_tiling_on_sc=False)


def _f32(shape):
    return jax.ShapeDtypeStruct(shape, jnp.float32)


def _wait(src, dst, sem):
    pltpu.make_async_copy(src, dst, sem).wait()


@functools.partial(
    pl.kernel,
    out_type=(_f32((EPAD, 16)), _f32((2, NT, 16))),
    mesh=_mesh,
    compiler_params=_sc_params,
    scratch_types=[
        pltpu.VMEM((2, NJ, 2, 128), jnp.int32),
        pltpu.VMEM((2, B_C, 16), jnp.float32),
        pltpu.VMEM((2, B_C, 16), jnp.float32),
        pltpu.VMEM_SHARED((NT, 16), jnp.float32),
        pltpu.SemaphoreType.DMA,
        pltpu.SemaphoreType.DMA,
    ],
)
def _sc_pass_a(idx_hbm, as_hbm, ad_hbm, ex_hbm, den_hbm,
               idx_v, as_v, ad_v, den_sh, sem0, sem1):
    cid = lax.axis_index("c")
    sid = lax.axis_index("s")
    wid = cid * 16 + sid
    sems = (sem0, sem1)

    z16 = jnp.zeros((16,), jnp.float32)

    def _zb(i, _):
        as_v[0, i, :] = z16
        return 0

    lax.fori_loop(0, B_C, _zb, 0, unroll=8)
    for r in range(RPT // B_C):
        pltpu.sync_copy(as_v.at[0],
                        den_sh.at[pl.ds(sid * RPT + r * B_C, B_C)])
    plsc.subcore_barrier()

    base_w = wid * (C_W * B_C)

    def _load_and_gather(t, b):
        g0 = (base_w + t * B_C) // 128
        pltpu.sync_copy(idx_hbm.at[pl.ds(g0, NJ)], idx_v.at[b])
        for j in range(NJ):
            pltpu.async_copy(as_hbm.at[idx_v.at[b, j, 0]],
                             as_v.at[b, pl.ds(j * 128, 128)], sems[b])
            pltpu.async_copy(ad_hbm.at[idx_v.at[b, j, 1]],
                             ad_v.at[b, pl.ds(j * 128, 128)], sems[b])

    def _wait_gathers(b):
        for j in range(NJ):
            _wait(as_hbm.at[idx_v.at[b, j, 0]],
                  as_v.at[b, pl.ds(j * 128, 128)], sems[b])
            _wait(ad_hbm.at[idx_v.at[b, j, 1]],
                  ad_v.at[b, pl.ds(j * 128, 128)], sems[b])

    def _scatter(t, b):
        base = base_w + t * B_C
        pltpu.sync_copy(as_v.at[b], ex_hbm.at[pl.ds(base, B_C)])
        for j in range(NJ):
            pltpu.sync_copy(as_v.at[b, pl.ds(j * 128, 128)],
                            den_sh.at[idx_v.at[b, j, 1]], add=True)

    _load_and_gather(0, 0)

    def _outer(tt, _):
        for b in range(2):
            t = 2 * tt + b

            @pl.when(t + 1 < C_W)
            def _():
                _load_and_gather(t + 1, b ^ 1)

            _wait_gathers(b)

            def _edge(i, _):
                e = as_v[b, i, :] + ad_v[b, i, :]
                e = jnp.where(e > 0, e, 0.2 * e)
                as_v[b, i, :] = jnp.exp(e)
                return 0

            lax.fori_loop(0, B_C, _edge, 0, unroll=16)
            _scatter(t, b)
        return 0

    lax.fori_loop(0, C_W // 2, _outer, 0)

    plsc.subcore_barrier()
    pltpu.sync_copy(den_sh.at[pl.ds(sid * RPT, RPT)],
                    den_hbm.at[cid, pl.ds(sid * RPT, RPT)])


def _make_sc_pass_b(D, expand_pairs, nj):
    b_c = nj * 128
    c_w = (C_W * B_C) // b_c

    @functools.partial(
        pl.kernel,
        out_type=_f32((2, NT, D)),
        mesh=_mesh,
        compiler_params=_sc_params,
        scratch_types=[
            pltpu.VMEM((2, nj, 2, 128), jnp.int32),
            pltpu.VMEM((2, b_c, 16), jnp.float32),
            pltpu.VMEM((2, b_c, 16), jnp.float32),
            pltpu.VMEM((2, b_c, D), jnp.float32),
            pltpu.VMEM_SHARED((NT, D), jnp.float32),
            pltpu.SemaphoreType.DMA,
            pltpu.SemaphoreType.DMA,
        ],
    )
    def _sc_pass_b(idx_hbm, ex_hbm, h_hbm, den_hbm, out_hbm,
                   idx_v, ex_v, den_v, h_v, out_sh,
                   sem0, sem1):
        cid = lax.axis_index("c")
        sid = lax.axis_index("s")
        wid = cid * 16 + sid
        sems = (sem0, sem1)

        z16 = jnp.zeros((16,), jnp.float32)

        def _zb(i, _):
            for k in range(D // 16):
                h_v[0, i, pl.ds(k * 16, 16)] = z16
            return 0

        lax.fori_loop(0, 128, _zb, 0, unroll=8)
        for r in range(RPT // 128):
            pltpu.sync_copy(h_v.at[0, pl.ds(0, 128)],
                            out_sh.at[pl.ds(sid * RPT + r * 128, 128)])
        plsc.subcore_barrier()

        if expand_pairs:
            lane_hi = lax.iota(jnp.int32, 16) >= 8

        base_w = wid * (c_w * b_c)

        def _load_and_gather(t, b):
            base = base_w + t * b_c
            g0 = base // 128
            pltpu.sync_copy(idx_hbm.at[pl.ds(g0, nj)], idx_v.at[b])
            pltpu.async_copy(ex_hbm.at[pl.ds(base, b_c)], ex_v.at[b],
                             sems[b])
            for j in range(nj):
                pltpu.async_copy(h_hbm.at[idx_v.at[b, j, 0]],
                                 h_v.at[b, pl.ds(j * 128, 128)], sems[b])
                pltpu.async_copy(den_hbm.at[idx_v.at[b, j, 1]],
                                 den_v.at[b, pl.ds(j * 128, 128)], sems[b])

        def _wait_gathers(t, b):
            base = base_w + t * b_c
            _wait(ex_hbm.at[pl.ds(base, b_c)], ex_v.at[b], sems[b])
            for j in range(nj):
                _wait(h_hbm.at[idx_v.at[b, j, 0]],
                      h_v.at[b, pl.ds(j * 128, 128)], sems[b])
                _wait(den_hbm.at[idx_v.at[b, j, 1]],
                      den_v.at[b, pl.ds(j * 128, 128)], sems[b])

        def _scatter(b):
            for j in range(nj):
                pltpu.sync_copy(h_v.at[b, pl.ds(j * 128, 128)],
                                out_sh.at[idx_v.at[b, j, 1]], add=True)

        _load_and_gather(0, 0)

        def _outer(tt, _):
            for b in range(2):
                t = 2 * tt + b

                @pl.when(t + 1 < c_w)
                def _():
                    _load_and_gather(t + 1, b ^ 1)

                _wait_gathers(t, b)

                def _edge(i, _):
                    alpha = ex_v[b, i, :] / (den_v[b, i, :] + 1e-16)
                    for k in range(D // 16):
                        if expand_pairs:
                            a = jnp.where(lane_hi, alpha[2 * k + 1],
                                          alpha[2 * k])
                        else:
                            a = alpha
                        h_v[b, i, pl.ds(k * 16, 16)] = (
                            h_v[b, i, pl.ds(k * 16, 16)] * a)
                    return 0

                lax.fori_loop(0, b_c, _edge, 0, unroll=8)
                _scatter(b)
            return 0

        lax.fori_loop(0, c_w // 2, _outer, 0)

        plsc.subcore_barrier()
        pltpu.sync_copy(out_sh.at[pl.ds(sid * RPT, RPT)],
                        out_hbm.at[cid, pl.ds(sid * RPT, RPT)])

    return _sc_pass_b


_sc_pass_b1 = _make_sc_pass_b(D1, True, 2)
_sc_pass_b2 = _make_sc_pass_b(D2, False, 1)


_BR = 256


def _proj1_body(x_ref, w_ref, was_ref, wad_ref, h_ref, as_ref, ad_ref):
    x = x_ref[...]
    h_ref[...] = jnp.dot(x, w_ref[...], preferred_element_type=jnp.float32)
    as_ref[...] = jnp.dot(x, was_ref[...], preferred_element_type=jnp.float32)
    ad_ref[...] = jnp.dot(x, wad_ref[...], preferred_element_type=jnp.float32)


def _proj1(xp, W1, Was16, Wad16):
    return pl.pallas_call(
        _proj1_body,
        grid=(NT // _BR,),
        in_specs=[
            pl.BlockSpec((_BR, D_IN), lambda i: (i, 0)),
            pl.BlockSpec((D_IN, D1), lambda i: (0, 0)),
            pl.BlockSpec((D_IN, 16), lambda i: (0, 0)),
            pl.BlockSpec((D_IN, 16), lambda i: (0, 0)),
        ],
        out_specs=[
            pl.BlockSpec((_BR, D1), lambda i: (i, 0)),
            pl.BlockSpec((_BR, 16), lambda i: (i, 0)),
            pl.BlockSpec((_BR, 16), lambda i: (i, 0)),
        ],
        out_shape=[_f32((NT, D1)), _f32((NT, 16)), _f32((NT, 16))],
    )(xp, W1, Was16, Wad16)


def _proj2_body(p0_ref, p1_ref, b_ref, w_ref, was_ref, wad_ref,
                h_ref, as_ref, ad_ref):
    h1e = p0_ref[0] + p1_ref[0] + b_ref[...]
    h1e = jnp.where(h1e > 0, h1e, jnp.exp(h1e) - 1.0)
    h_ref[...] = jnp.dot(h1e, w_ref[...], preferred_element_type=jnp.float32)
    as_ref[...] = jnp.dot(h1e, was_ref[...],
                          preferred_element_type=jnp.float32)
    ad_ref[...] = jnp.dot(h1e, wad_ref[...],
                          preferred_element_type=jnp.float32)


def _proj2(out1p, b1, W2, Was16, Wad16):
    return pl.pallas_call(
        _proj2_body,
        grid=(NT // _BR,),
        in_specs=[
            pl.BlockSpec((1, _BR, D1), lambda i: (0, i, 0)),
            pl.BlockSpec((1, _BR, D1), lambda i: (1, i, 0)),
            pl.BlockSpec((1, D1), lambda i: (0, 0)),
            pl.BlockSpec((D1, D2), lambda i: (0, 0)),
            pl.BlockSpec((D1, 16), lambda i: (0, 0)),
            pl.BlockSpec((D1, 16), lambda i: (0, 0)),
        ],
        out_specs=[
            pl.BlockSpec((_BR, D2), lambda i: (i, 0)),
            pl.BlockSpec((_BR, 16), lambda i: (i, 0)),
            pl.BlockSpec((_BR, 16), lambda i: (i, 0)),
        ],
        out_shape=[_f32((NT, D2)), _f32((NT, 16)), _f32((NT, 16))],
    )(out1p, out1p, b1, W2, Was16, Wad16)


def _comb_body(p0_ref, p1_ref, o_ref):
    o_ref[...] = p0_ref[0] + p1_ref[0]


def _combine(parts):
    D = parts.shape[-1]
    return pl.pallas_call(
        _comb_body,
        grid=(NT // _BR,),
        in_specs=[
            pl.BlockSpec((1, _BR, D), lambda i: (0, i, 0)),
            pl.BlockSpec((1, _BR, D), lambda i: (1, i, 0)),
        ],
        out_specs=pl.BlockSpec((_BR, D), lambda i: (i, 0)),
        out_shape=_f32((NT, D)),
    )(parts, parts)


def _final_body(p0_ref, p1_ref, b_ref, o_ref):
    o_ref[...] = p0_ref[0] + p1_ref[0] + b_ref[...]


def _final(parts, b2):
    return pl.pallas_call(
        _final_body,
        grid=(NT // _BR,),
        in_specs=[
            pl.BlockSpec((1, _BR, D2), lambda i: (0, i, 0)),
            pl.BlockSpec((1, _BR, D2), lambda i: (1, i, 0)),
            pl.BlockSpec((1, D2), lambda i: (0, 0)),
        ],
        out_specs=pl.BlockSpec((_BR, D2), lambda i: (i, 0)),
        out_shape=_f32((NT, D2)),
    )(parts, parts, b2)


def kernel(x, edge_index, W1, a_src1, a_dst1, b1, W2, a_src2, a_dst2, b2):
    loops = jnp.arange(N, dtype=edge_index.dtype)
    src = jnp.concatenate([edge_index[0], loops])
    dst = jnp.concatenate([edge_index[1], loops])
    pad = EPAD - E_TOT
    sent = (N + jnp.arange(pad, dtype=jnp.int32) % (NT - N)).astype(
        edge_index.dtype)
    src = jnp.concatenate([src, sent])
    dst = jnp.concatenate([dst, sent])
    idx2 = jnp.stack([src.reshape(-1, 128), dst.reshape(-1, 128)], axis=1)

    xp = jnp.pad(x, ((0, NT - N), (0, 0)))

    Was1 = (W1.reshape(D_IN, H1, C1) * a_src1[None]).sum(-1)
    Wad1 = (W1.reshape(D_IN, H1, C1) * a_dst1[None]).sum(-1)
    Was1_16 = jnp.concatenate([Was1, Was1], axis=1)
    Wad1_16 = jnp.concatenate([Wad1, Wad1], axis=1)
    Was2_16 = jnp.tile((W2 @ a_src2[0])[:, None], (1, 16))
    Wad2_16 = jnp.tile((W2 @ a_dst2[0])[:, None], (1, 16))

    h1, as1, ad1 = _proj1(xp, W1, Was1_16, Wad1_16)
    ex1, den1p = _sc_pass_a(idx2, as1, ad1)
    den1 = _combine(den1p)
    out1p = _sc_pass_b1(idx2, ex1, h1, den1)

    h2, as2, ad2 = _proj2(out1p, b1[None, :], W2, Was2_16, Wad2_16)
    ex2, den2p = _sc_pass_a(idx2, as2, ad2)
    den2 = _combine(den2p)
    out2p = _sc_pass_b2(idx2, ex2, h2, den2)

    out = _final(out2p, b2[None, :])
    return out[:N]


# speedup vs baseline: 1.2089x; 1.0282x over previous
import functools

import jax
import jax.numpy as jnp
from jax import lax
from jax.experimental import pallas as pl
from jax.experimental.pallas import tpu as pltpu
from jax.experimental.pallas import tpu_sc as plsc

N = 10000
D_IN = 128
H1 = 8
C1 = 8
D1 = H1 * C1
D2 = 128

NT = 10240
NW = 32
NJ = 2
B_C = NJ * 128
E_TOT = 320000 + N
C_W = 42
EPAD = NW * B_C * C_W
NJA = 6
B_A = NJA * 128
C_A = (C_W * B_C) // B_A
RPT = NT // 16

_mesh = plsc.VectorSubcoreMesh(core_axis_name="c", subcore_axis_name="s",
                               num_cores=2, num_subcores=16)
_sc_params = pltpu.CompilerParams(use_tc_tiling_on_sc=False)


def _f32(shape):
    return jax.ShapeDtypeStruct(shape, jnp.float32)


def _wait(src, dst, sem):
    pltpu.make_async_copy(src, dst, sem).wait()


@functools.partial(
    pl.kernel,
    out_type=(_f32((EPAD, 16)), _f32((2, NT, 16))),
    mesh=_mesh,
    compiler_params=_sc_params,
    scratch_types=[
        pltpu.VMEM((2, NJA, 2, 128), jnp.int32),
        pltpu.VMEM((2, B_A, 16), jnp.float32),
        pltpu.VMEM((2, B_A, 16), jnp.float32),
        pltpu.VMEM_SHARED((NT, 16), jnp.float32),
        pltpu.SemaphoreType.DMA,
        pltpu.SemaphoreType.DMA,
    ],
)
def _sc_pass_a(idx_hbm, as_hbm, ad_hbm, ex_hbm, den_hbm,
               idx_v, as_v, ad_v, den_sh, sem0, sem1):
    cid = lax.axis_index("c")
    sid = lax.axis_index("s")
    wid = cid * 16 + sid
    sems = (sem0, sem1)

    z16 = jnp.zeros((16,), jnp.float32)

    def _zb(i, _):
        as_v[0, i, :] = z16
        return 0

    lax.fori_loop(0, B_A, _zb, 0, unroll=8)
    pltpu.sync_copy(as_v.at[0, pl.ds(0, RPT)],
                    den_sh.at[pl.ds(sid * RPT, RPT)])
    plsc.subcore_barrier()

    base_w = wid * (C_A * B_A)

    def _load_and_gather(t, b):
        g0 = (base_w + t * B_A) // 128
        pltpu.sync_copy(idx_hbm.at[pl.ds(g0, NJA)], idx_v.at[b])
        for j in range(NJA):
            pltpu.async_copy(as_hbm.at[idx_v.at[b, j, 0]],
                             as_v.at[b, pl.ds(j * 128, 128)], sems[b])
            pltpu.async_copy(ad_hbm.at[idx_v.at[b, j, 1]],
                             ad_v.at[b, pl.ds(j * 128, 128)], sems[b])

    def _wait_gathers(b):
        for j in range(NJA):
            _wait(as_hbm.at[idx_v.at[b, j, 0]],
                  as_v.at[b, pl.ds(j * 128, 128)], sems[b])
            _wait(ad_hbm.at[idx_v.at[b, j, 1]],
                  ad_v.at[b, pl.ds(j * 128, 128)], sems[b])

    def _scatter(t, b):
        base = base_w + t * B_A
        pltpu.sync_copy(as_v.at[b], ex_hbm.at[pl.ds(base, B_A)])
        for j in range(NJA):
            pltpu.sync_copy(as_v.at[b, pl.ds(j * 128, 128)],
                            den_sh.at[idx_v.at[b, j, 1]], add=True)

    _load_and_gather(0, 0)

    def _outer(tt, _):
        for b in range(2):
            t = 2 * tt + b

            @pl.when(t + 1 < C_A)
            def _():
                _load_and_gather(t + 1, b ^ 1)

            _wait_gathers(b)

            def _edge(i, _):
                e = as_v[b, i, :] + ad_v[b, i, :]
                e = jnp.where(e > 0, e, 0.2 * e)
                as_v[b, i, :] = jnp.exp(e)
                return 0

            lax.fori_loop(0, B_A, _edge, 0, unroll=16)
            _scatter(t, b)
        return 0

    lax.fori_loop(0, C_A // 2, _outer, 0)

    plsc.subcore_barrier()
    pltpu.sync_copy(den_sh.at[pl.ds(sid * RPT, RPT)],
                    den_hbm.at[cid, pl.ds(sid * RPT, RPT)])


def _make_sc_pass_b(D, expand_pairs, nj):
    b_c = nj * 128
    c_w = (C_W * B_C) // b_c

    @functools.partial(
        pl.kernel,
        out_type=_f32((2, NT, D)),
        mesh=_mesh,
        compiler_params=_sc_params,
        scratch_types=[
            pltpu.VMEM((2, nj, 2, 128), jnp.int32),
            pltpu.VMEM((2, b_c, 16), jnp.float32),
            pltpu.VMEM((2, b_c, 16), jnp.float32),
            pltpu.VMEM((2, b_c, D), jnp.float32),
            pltpu.VMEM_SHARED((NT, D), jnp.float32),
            pltpu.SemaphoreType.DMA,
            pltpu.SemaphoreType.DMA,
        ],
    )
    def _sc_pass_b(idx_hbm, ex_hbm, h_hbm, den_hbm, out_hbm,
                   idx_v, ex_v, den_v, h_v, out_sh,
                   sem0, sem1):
        cid = lax.axis_index("c")
        sid = lax.axis_index("s")
        wid = cid * 16 + sid
        sems = (sem0, sem1)

        z16 = jnp.zeros((16,), jnp.float32)

        def _zb(i, _):
            for k in range(D // 16):
                h_v[0, i, pl.ds(k * 16, 16)] = z16
            return 0

        lax.fori_loop(0, 128, _zb, 0, unroll=8)
        for r in range(RPT // 128):
            pltpu.sync_copy(h_v.at[0, pl.ds(0, 128)],
                            out_sh.at[pl.ds(sid * RPT + r * 128, 128)])
        plsc.subcore_barrier()

        if expand_pairs:
            lane_hi = lax.iota(jnp.int32, 16) >= 8

        base_w = wid * (c_w * b_c)

        def _load_and_gather(t, b):
            base = base_w + t * b_c
            g0 = base // 128
            pltpu.sync_copy(idx_hbm.at[pl.ds(g0, nj)], idx_v.at[b])
            pltpu.async_copy(ex_hbm.at[pl.ds(base, b_c)], ex_v.at[b],
                             sems[b])
            for j in range(nj):
                pltpu.async_copy(h_hbm.at[idx_v.at[b, j, 0]],
                                 h_v.at[b, pl.ds(j * 128, 128)], sems[b])
                pltpu.async_copy(den_hbm.at[idx_v.at[b, j, 1]],
                                 den_v.at[b, pl.ds(j * 128, 128)], sems[b])

        def _wait_gathers(t, b):
            base = base_w + t * b_c
            _wait(ex_hbm.at[pl.ds(base, b_c)], ex_v.at[b], sems[b])
            for j in range(nj):
                _wait(h_hbm.at[idx_v.at[b, j, 0]],
                      h_v.at[b, pl.ds(j * 128, 128)], sems[b])
                _wait(den_hbm.at[idx_v.at[b, j, 1]],
                      den_v.at[b, pl.ds(j * 128, 128)], sems[b])

        def _scatter(b):
            for j in range(nj):
                pltpu.sync_copy(h_v.at[b, pl.ds(j * 128, 128)],
                                out_sh.at[idx_v.at[b, j, 1]], add=True)

        _load_and_gather(0, 0)

        def _outer(tt, _):
            for b in range(2):
                t = 2 * tt + b

                @pl.when(t + 1 < c_w)
                def _():
                    _load_and_gather(t + 1, b ^ 1)

                _wait_gathers(t, b)

                def _edge(i, _):
                    alpha = ex_v[b, i, :] / (den_v[b, i, :] + 1e-16)
                    for k in range(D // 16):
                        if expand_pairs:
                            a = jnp.where(lane_hi, alpha[2 * k + 1],
                                          alpha[2 * k])
                        else:
                            a = alpha
                        h_v[b, i, pl.ds(k * 16, 16)] = (
                            h_v[b, i, pl.ds(k * 16, 16)] * a)
                    return 0

                lax.fori_loop(0, b_c, _edge, 0, unroll=8)
                _scatter(b)
            return 0

        lax.fori_loop(0, c_w // 2, _outer, 0)

        plsc.subcore_barrier()
        pltpu.sync_copy(out_sh.at[pl.ds(sid * RPT, RPT)],
                        out_hbm.at[cid, pl.ds(sid * RPT, RPT)])

    return _sc_pass_b


_sc_pass_b1 = _make_sc_pass_b(D1, True, 3)
_sc_pass_b2 = _make_sc_pass_b(D2, False, 1)


_BR = 256


def _proj1_body(x_ref, w_ref, was_ref, wad_ref, h_ref, as_ref, ad_ref):
    x = x_ref[...]
    h_ref[...] = jnp.dot(x, w_ref[...], preferred_element_type=jnp.float32)
    as_ref[...] = jnp.dot(x, was_ref[...], preferred_element_type=jnp.float32)
    ad_ref[...] = jnp.dot(x, wad_ref[...], preferred_element_type=jnp.float32)


def _proj1(xp, W1, Was16, Wad16):
    return pl.pallas_call(
        _proj1_body,
        grid=(NT // _BR,),
        in_specs=[
            pl.BlockSpec((_BR, D_IN), lambda i: (i, 0)),
            pl.BlockSpec((D_IN, D1), lambda i: (0, 0)),
            pl.BlockSpec((D_IN, 16), lambda i: (0, 0)),
            pl.BlockSpec((D_IN, 16), lambda i: (0, 0)),
        ],
        out_specs=[
            pl.BlockSpec((_BR, D1), lambda i: (i, 0)),
            pl.BlockSpec((_BR, 16), lambda i: (i, 0)),
            pl.BlockSpec((_BR, 16), lambda i: (i, 0)),
        ],
        out_shape=[_f32((NT, D1)), _f32((NT, 16)), _f32((NT, 16))],
    )(xp, W1, Was16, Wad16)


def _proj2_body(p0_ref, p1_ref, b_ref, w_ref, was_ref, wad_ref,
                h_ref, as_ref, ad_ref):
    h1e = p0_ref[0] + p1_ref[0] + b_ref[...]
    h1e = jnp.where(h1e > 0, h1e, jnp.exp(h1e) - 1.0)
    h_ref[...] = jnp.dot(h1e, w_ref[...], preferred_element_type=jnp.float32)
    as_ref[...] = jnp.dot(h1e, was_ref[...],
                          preferred_element_type=jnp.float32)
    ad_ref[...] = jnp.dot(h1e, wad_ref[...],
                          preferred_element_type=jnp.float32)


def _proj2(out1p, b1, W2, Was16, Wad16):
    return pl.pallas_call(
        _proj2_body,
        grid=(NT // _BR,),
        in_specs=[
            pl.BlockSpec((1, _BR, D1), lambda i: (0, i, 0)),
            pl.BlockSpec((1, _BR, D1), lambda i: (1, i, 0)),
            pl.BlockSpec((1, D1), lambda i: (0, 0)),
            pl.BlockSpec((D1, D2), lambda i: (0, 0)),
            pl.BlockSpec((D1, 16), lambda i: (0, 0)),
            pl.BlockSpec((D1, 16), lambda i: (0, 0)),
        ],
        out_specs=[
            pl.BlockSpec((_BR, D2), lambda i: (i, 0)),
            pl.BlockSpec((_BR, 16), lambda i: (i, 0)),
            pl.BlockSpec((_BR, 16), lambda i: (i, 0)),
        ],
        out_shape=[_f32((NT, D2)), _f32((NT, 16)), _f32((NT, 16))],
    )(out1p, out1p, b1, W2, Was16, Wad16)


def _comb_body(p0_ref, p1_ref, o_ref):
    o_ref[...] = p0_ref[0] + p1_ref[0]


def _combine(parts):
    D = parts.shape[-1]
    return pl.pallas_call(
        _comb_body,
        grid=(NT // _BR,),
        in_specs=[
            pl.BlockSpec((1, _BR, D), lambda i: (0, i, 0)),
            pl.BlockSpec((1, _BR, D), lambda i: (1, i, 0)),
        ],
        out_specs=pl.BlockSpec((_BR, D), lambda i: (i, 0)),
        out_shape=_f32((NT, D)),
    )(parts, parts)


def _final_body(p0_ref, p1_ref, b_ref, o_ref):
    o_ref[...] = p0_ref[0] + p1_ref[0] + b_ref[...]


def _final(parts, b2):
    return pl.pallas_call(
        _final_body,
        grid=(NT // _BR,),
        in_specs=[
            pl.BlockSpec((1, _BR, D2), lambda i: (0, i, 0)),
            pl.BlockSpec((1, _BR, D2), lambda i: (1, i, 0)),
            pl.BlockSpec((1, D2), lambda i: (0, 0)),
        ],
        out_specs=pl.BlockSpec((_BR, D2), lambda i: (i, 0)),
        out_shape=_f32((NT, D2)),
    )(parts, parts, b2)


def kernel(x, edge_index, W1, a_src1, a_dst1, b1, W2, a_src2, a_dst2, b2):
    loops = jnp.arange(N, dtype=edge_index.dtype)
    src = jnp.concatenate([edge_index[0], loops])
    dst = jnp.concatenate([edge_index[1], loops])
    pad = EPAD - E_TOT
    sent = (N + jnp.arange(pad, dtype=jnp.int32) % (NT - N)).astype(
        edge_index.dtype)
    src = jnp.concatenate([src, sent])
    dst = jnp.concatenate([dst, sent])
    idx2 = jnp.stack([src.reshape(-1, 128), dst.reshape(-1, 128)], axis=1)

    xp = jnp.pad(x, ((0, NT - N), (0, 0)))

    Was1 = (W1.reshape(D_IN, H1, C1) * a_src1[None]).sum(-1)
    Wad1 = (W1.reshape(D_IN, H1, C1) * a_dst1[None]).sum(-1)
    Was1_16 = jnp.concatenate([Was1, Was1], axis=1)
    Wad1_16 = jnp.concatenate([Wad1, Wad1], axis=1)
    Was2_16 = jnp.tile((W2 @ a_src2[0])[:, None], (1, 16))
    Wad2_16 = jnp.tile((W2 @ a_dst2[0])[:, None], (1, 16))

    h1, as1, ad1 = _proj1(xp, W1, Was1_16, Wad1_16)
    ex1, den1p = _sc_pass_a(idx2, as1, ad1)
    den1 = _combine(den1p)
    out1p = _sc_pass_b1(idx2, ex1, h1, den1)

    h2, as2, ad2 = _proj2(out1p, b1[None, :], W2, Was2_16, Wad2_16)
    ex2, den2p = _sc_pass_a(idx2, as2, ad2)
    den2 = _combine(den2p)
    out2p = _sc_pass_b2(idx2, ex2, h2, den2)

    out = _final(out2p, b2[None, :])
    return out[:N]


# speedup vs baseline: 1.2138x; 1.0041x over previous
import functools

import jax
import jax.numpy as jnp
from jax import lax
from jax.experimental import pallas as pl
from jax.experimental.pallas import tpu as pltpu
from jax.experimental.pallas import tpu_sc as plsc

N = 10000
D_IN = 128
H1 = 8
C1 = 8
D1 = H1 * C1
D2 = 128

NT = 10240
NW = 32
NJ = 2
B_C = NJ * 128
E_TOT = 320000 + N
C_W = 42
EPAD = NW * B_C * C_W
NJA = 6
B_A = NJA * 128
C_A = (C_W * B_C) // B_A
RPT = NT // 16

_mesh = plsc.VectorSubcoreMesh(core_axis_name="c", subcore_axis_name="s",
                               num_cores=2, num_subcores=16)
_sc_params = pltpu.CompilerParams(use_tc_tiling_on_sc=False)


def _f32(shape):
    return jax.ShapeDtypeStruct(shape, jnp.float32)


def _wait(src, dst, sem):
    pltpu.make_async_copy(src, dst, sem).wait()


@functools.partial(
    pl.kernel,
    out_type=(_f32((EPAD, 16)), _f32((2, NT, 16))),
    mesh=_mesh,
    compiler_params=_sc_params,
    scratch_types=[
        pltpu.VMEM((2, NJA, 2, 128), jnp.int32),
        pltpu.VMEM((2, B_A, 16), jnp.float32),
        pltpu.VMEM((2, B_A, 16), jnp.float32),
        pltpu.VMEM_SHARED((NT, 16), jnp.float32),
        pltpu.SemaphoreType.DMA,
        pltpu.SemaphoreType.DMA,
    ],
)
def _sc_pass_a(idx_hbm, as_hbm, ad_hbm, ex_hbm, den_hbm,
               idx_v, as_v, ad_v, den_sh, sem0, sem1):
    cid = lax.axis_index("c")
    sid = lax.axis_index("s")
    wid = cid * 16 + sid
    sems = (sem0, sem1)

    z16 = jnp.zeros((16,), jnp.float32)

    def _zb(i, _):
        as_v[0, i, :] = z16
        return 0

    lax.fori_loop(0, B_A, _zb, 0, unroll=8)
    pltpu.sync_copy(as_v.at[0, pl.ds(0, RPT)],
                    den_sh.at[pl.ds(sid * RPT, RPT)])
    plsc.subcore_barrier()

    base_w = wid * (C_A * B_A)

    def _load_and_gather(t, b):
        g0 = (base_w + t * B_A) // 128
        pltpu.sync_copy(idx_hbm.at[pl.ds(g0, NJA)], idx_v.at[b])
        for j in range(NJA):
            pltpu.async_copy(as_hbm.at[idx_v.at[b, j, 0]],
                             as_v.at[b, pl.ds(j * 128, 128)], sems[b])
            pltpu.async_copy(ad_hbm.at[idx_v.at[b, j, 1]],
                             ad_v.at[b, pl.ds(j * 128, 128)], sems[b])

    def _wait_gathers(b):
        for j in range(NJA):
            _wait(as_hbm.at[idx_v.at[b, j, 0]],
                  as_v.at[b, pl.ds(j * 128, 128)], sems[b])
            _wait(ad_hbm.at[idx_v.at[b, j, 1]],
                  ad_v.at[b, pl.ds(j * 128, 128)], sems[b])

    def _scatter(t, b):
        base = base_w + t * B_A
        pltpu.sync_copy(as_v.at[b], ex_hbm.at[pl.ds(base, B_A)])
        for j in range(NJA):
            pltpu.sync_copy(as_v.at[b, pl.ds(j * 128, 128)],
                            den_sh.at[idx_v.at[b, j, 1]], add=True)

    _load_and_gather(0, 0)

    def _outer(tt, _):
        for b in range(2):
            t = 2 * tt + b

            @pl.when(t + 1 < C_A)
            def _():
                _load_and_gather(t + 1, b ^ 1)

            _wait_gathers(b)

            def _edge(i, _):
                e = as_v[b, i, :] + ad_v[b, i, :]
                e = jnp.where(e > 0, e, 0.2 * e)
                as_v[b, i, :] = jnp.exp(e)
                return 0

            lax.fori_loop(0, B_A, _edge, 0, unroll=16)
            _scatter(t, b)
        return 0

    lax.fori_loop(0, C_A // 2, _outer, 0)

    plsc.subcore_barrier()
    pltpu.sync_copy(den_sh.at[pl.ds(sid * RPT, RPT)],
                    den_hbm.at[cid, pl.ds(sid * RPT, RPT)])


def _make_sc_pass_b(D, expand_pairs, nj):
    b_c = nj * 128
    c_w = (C_W * B_C) // b_c

    @functools.partial(
        pl.kernel,
        out_type=_f32((2, NT, D)),
        mesh=_mesh,
        compiler_params=_sc_params,
        scratch_types=[
            pltpu.VMEM((2, nj, 2, 128), jnp.int32),
            pltpu.VMEM((2, b_c, 16), jnp.float32),
            pltpu.VMEM((2, b_c, 16), jnp.float32),
            pltpu.VMEM((2, b_c, D), jnp.float32),
            pltpu.VMEM_SHARED((NT, D), jnp.float32),
            pltpu.SemaphoreType.DMA,
            pltpu.SemaphoreType.DMA,
            pltpu.SemaphoreType.DMA,
            pltpu.SemaphoreType.DMA,
        ],
    )
    def _sc_pass_b(idx_hbm, ex_hbm, h_hbm, den_hbm, out_hbm,
                   idx_v, ex_v, den_v, h_v, out_sh,
                   sem0, sem1, sem2, sem3):
        cid = lax.axis_index("c")
        sid = lax.axis_index("s")
        wid = cid * 16 + sid
        sems = (sem0, sem1)
        ssems = (sem2, sem3)

        z16 = jnp.zeros((16,), jnp.float32)

        def _zb(i, _):
            for k in range(D // 16):
                h_v[0, i, pl.ds(k * 16, 16)] = z16
            return 0

        lax.fori_loop(0, 128, _zb, 0, unroll=8)
        for r in range(RPT // 128):
            pltpu.sync_copy(h_v.at[0, pl.ds(0, 128)],
                            out_sh.at[pl.ds(sid * RPT + r * 128, 128)])
        plsc.subcore_barrier()

        if expand_pairs:
            lane_hi = lax.iota(jnp.int32, 16) >= 8

        base_w = wid * (c_w * b_c)

        def _load_and_gather(t, b):
            base = base_w + t * b_c
            g0 = base // 128
            pltpu.sync_copy(idx_hbm.at[pl.ds(g0, nj)], idx_v.at[b])
            pltpu.async_copy(ex_hbm.at[pl.ds(base, b_c)], ex_v.at[b],
                             sems[b])
            for j in range(nj):
                pltpu.async_copy(h_hbm.at[idx_v.at[b, j, 0]],
                                 h_v.at[b, pl.ds(j * 128, 128)], sems[b])
                pltpu.async_copy(den_hbm.at[idx_v.at[b, j, 1]],
                                 den_v.at[b, pl.ds(j * 128, 128)], sems[b])

        def _wait_gathers(t, b):
            base = base_w + t * b_c
            _wait(ex_hbm.at[pl.ds(base, b_c)], ex_v.at[b], sems[b])
            for j in range(nj):
                _wait(h_hbm.at[idx_v.at[b, j, 0]],
                      h_v.at[b, pl.ds(j * 128, 128)], sems[b])
                _wait(den_hbm.at[idx_v.at[b, j, 1]],
                      den_v.at[b, pl.ds(j * 128, 128)], sems[b])

        def _scatter(b):
            for j in range(nj):
                pltpu.async_copy(h_v.at[b, pl.ds(j * 128, 128)],
                                 out_sh.at[idx_v.at[b, j, 1]], ssems[b],
                                 add=True)

        def _wait_scatter(b):
            for j in range(nj):
                _wait(h_v.at[b, pl.ds(j * 128, 128)],
                      out_sh.at[idx_v.at[b, j, 1]], ssems[b])

        _load_and_gather(0, 0)

        def _outer(tt, _):
            for b in range(2):
                t = 2 * tt + b

                if b == 0:
                    @pl.when(tt >= 1)
                    def _():
                        _wait_scatter(b ^ 1)
                else:
                    _wait_scatter(b ^ 1)

                @pl.when(t + 1 < c_w)
                def _():
                    _load_and_gather(t + 1, b ^ 1)

                _wait_gathers(t, b)

                def _edge(i, _):
                    alpha = ex_v[b, i, :] / (den_v[b, i, :] + 1e-16)
                    for k in range(D // 16):
                        if expand_pairs:
                            a = jnp.where(lane_hi, alpha[2 * k + 1],
                                          alpha[2 * k])
                        else:
                            a = alpha
                        h_v[b, i, pl.ds(k * 16, 16)] = (
                            h_v[b, i, pl.ds(k * 16, 16)] * a)
                    return 0

                lax.fori_loop(0, b_c, _edge, 0, unroll=8)
                _scatter(b)
            return 0

        lax.fori_loop(0, c_w // 2, _outer, 0)
        _wait_scatter(1)

        plsc.subcore_barrier()
        pltpu.sync_copy(out_sh.at[pl.ds(sid * RPT, RPT)],
                        out_hbm.at[cid, pl.ds(sid * RPT, RPT)])

    return _sc_pass_b


_sc_pass_b1 = _make_sc_pass_b(D1, True, 3)
_sc_pass_b2 = _make_sc_pass_b(D2, False, 1)


_BR = 256


def _proj1_body(x_ref, w_ref, was_ref, wad_ref, h_ref, as_ref, ad_ref):
    x = x_ref[...]
    h_ref[...] = jnp.dot(x, w_ref[...], preferred_element_type=jnp.float32)
    as_ref[...] = jnp.dot(x, was_ref[...], preferred_element_type=jnp.float32)
    ad_ref[...] = jnp.dot(x, wad_ref[...], preferred_element_type=jnp.float32)


def _proj1(xp, W1, Was16, Wad16):
    return pl.pallas_call(
        _proj1_body,
        grid=(NT // _BR,),
        in_specs=[
            pl.BlockSpec((_BR, D_IN), lambda i: (i, 0)),
            pl.BlockSpec((D_IN, D1), lambda i: (0, 0)),
            pl.BlockSpec((D_IN, 16), lambda i: (0, 0)),
            pl.BlockSpec((D_IN, 16), lambda i: (0, 0)),
        ],
        out_specs=[
            pl.BlockSpec((_BR, D1), lambda i: (i, 0)),
            pl.BlockSpec((_BR, 16), lambda i: (i, 0)),
            pl.BlockSpec((_BR, 16), lambda i: (i, 0)),
        ],
        out_shape=[_f32((NT, D1)), _f32((NT, 16)), _f32((NT, 16))],
    )(xp, W1, Was16, Wad16)


def _proj2_body(p0_ref, p1_ref, b_ref, w_ref, was_ref, wad_ref,
                h_ref, as_ref, ad_ref):
    h1e = p0_ref[0] + p1_ref[0] + b_ref[...]
    h1e = jnp.where(h1e > 0, h1e, jnp.exp(h1e) - 1.0)
    h_ref[...] = jnp.dot(h1e, w_ref[...], preferred_element_type=jnp.float32)
    as_ref[...] = jnp.dot(h1e, was_ref[...],
                          preferred_element_type=jnp.float32)
    ad_ref[...] = jnp.dot(h1e, wad_ref[...],
                          preferred_element_type=jnp.float32)


def _proj2(out1p, b1, W2, Was16, Wad16):
    return pl.pallas_call(
        _proj2_body,
        grid=(NT // _BR,),
        in_specs=[
            pl.BlockSpec((1, _BR, D1), lambda i: (0, i, 0)),
            pl.BlockSpec((1, _BR, D1), lambda i: (1, i, 0)),
            pl.BlockSpec((1, D1), lambda i: (0, 0)),
            pl.BlockSpec((D1, D2), lambda i: (0, 0)),
            pl.BlockSpec((D1, 16), lambda i: (0, 0)),
            pl.BlockSpec((D1, 16), lambda i: (0, 0)),
        ],
        out_specs=[
            pl.BlockSpec((_BR, D2), lambda i: (i, 0)),
            pl.BlockSpec((_BR, 16), lambda i: (i, 0)),
            pl.BlockSpec((_BR, 16), lambda i: (i, 0)),
        ],
        out_shape=[_f32((NT, D2)), _f32((NT, 16)), _f32((NT, 16))],
    )(out1p, out1p, b1, W2, Was16, Wad16)


def _comb_body(p0_ref, p1_ref, o_ref):
    o_ref[...] = p0_ref[0] + p1_ref[0]


def _combine(parts):
    D = parts.shape[-1]
    return pl.pallas_call(
        _comb_body,
        grid=(NT // _BR,),
        in_specs=[
            pl.BlockSpec((1, _BR, D), lambda i: (0, i, 0)),
            pl.BlockSpec((1, _BR, D), lambda i: (1, i, 0)),
        ],
        out_specs=pl.BlockSpec((_BR, D), lambda i: (i, 0)),
        out_shape=_f32((NT, D)),
    )(parts, parts)


def _final_body(p0_ref, p1_ref, b_ref, o_ref):
    o_ref[...] = p0_ref[0] + p1_ref[0] + b_ref[...]


def _final(parts, b2):
    return pl.pallas_call(
        _final_body,
        grid=(NT // _BR,),
        in_specs=[
            pl.BlockSpec((1, _BR, D2), lambda i: (0, i, 0)),
            pl.BlockSpec((1, _BR, D2), lambda i: (1, i, 0)),
            pl.BlockSpec((1, D2), lambda i: (0, 0)),
        ],
        out_specs=pl.BlockSpec((_BR, D2), lambda i: (i, 0)),
        out_shape=_f32((NT, D2)),
    )(parts, parts, b2)


def kernel(x, edge_index, W1, a_src1, a_dst1, b1, W2, a_src2, a_dst2, b2):
    loops = jnp.arange(N, dtype=edge_index.dtype)
    src = jnp.concatenate([edge_index[0], loops])
    dst = jnp.concatenate([edge_index[1], loops])
    pad = EPAD - E_TOT
    sent = (N + jnp.arange(pad, dtype=jnp.int32) % (NT - N)).astype(
        edge_index.dtype)
    src = jnp.concatenate([src, sent])
    dst = jnp.concatenate([dst, sent])
    idx2 = jnp.stack([src.reshape(-1, 128), dst.reshape(-1, 128)], axis=1)

    xp = jnp.pad(x, ((0, NT - N), (0, 0)))

    Was1 = (W1.reshape(D_IN, H1, C1) * a_src1[None]).sum(-1)
    Wad1 = (W1.reshape(D_IN, H1, C1) * a_dst1[None]).sum(-1)
    Was1_16 = jnp.concatenate([Was1, Was1], axis=1)
    Wad1_16 = jnp.concatenate([Wad1, Wad1], axis=1)
    Was2_16 = jnp.tile((W2 @ a_src2[0])[:, None], (1, 16))
    Wad2_16 = jnp.tile((W2 @ a_dst2[0])[:, None], (1, 16))

    h1, as1, ad1 = _proj1(xp, W1, Was1_16, Wad1_16)
    ex1, den1p = _sc_pass_a(idx2, as1, ad1)
    den1 = _combine(den1p)
    out1p = _sc_pass_b1(idx2, ex1, h1, den1)

    h2, as2, ad2 = _proj2(out1p, b1[None, :], W2, Was2_16, Wad2_16)
    ex2, den2p = _sc_pass_a(idx2, as2, ad2)
    den2 = _combine(den2p)
    out2p = _sc_pass_b2(idx2, ex2, h2, den2)

    out = _final(out2p, b2[None, :])
    return out[:N]
